# baseline (speedup 1.0000x reference)
import functools

import jax
import jax.numpy as jnp
from jax import lax
from jax.experimental import pallas as pl
from jax.experimental.pallas import tpu as pltpu
from jax.experimental.pallas import tpu_sc as plsc

_B = 16384
_D = 16
_C = 1000
_CPAD = 1024
_F = 8
_FD = _F * _D
_FDP = 128

_BM = 2048
_NC = 2
_NS = 16
_NW = _NC * _NS
_BW = _B // _NW
_CHUNK = 128
_NCHUNK = _BW // _CHUNK


def _coarse_body(cp_ref, xt_ref, out_ref):
    cp = cp_ref[...]
    xt = xt_ref[...]
    cp2 = -2.0 * cp
    dots2 = lax.dot_general(cp2, xt, (((1,), (0,)), ((), ())),
                            preferred_element_type=jnp.float32)
    row = lax.broadcasted_iota(jnp.int32, (_CPAD, 1), 0)
    b2p = jnp.sum(cp * cp, axis=1, keepdims=True) + jnp.where(
        row >= _C, jnp.float32(1e30), jnp.float32(0.0))
    a2 = jnp.sum(xt * xt, axis=0, keepdims=True)
    d2 = jnp.maximum((dots2 + a2) + b2p, 0.0)
    keys = (lax.bitcast_convert_type(d2, jnp.int32) & jnp.int32(~1023)) | row
    fkeys = lax.bitcast_convert_type(keys, jnp.float32)
    best = lax.bitcast_convert_type(jnp.min(fkeys, axis=0), jnp.int32)
    out_ref[0] = (best & jnp.int32(1023)).reshape(1, _BM)


@functools.cache
def _get_coarse_call(nb):
    return pl.pallas_call(
        _coarse_body,
        grid=(nb,),
        in_specs=[
            pl.BlockSpec((_CPAD, _D), lambda i: (0, 0)),
            pl.BlockSpec((_D, _BM), lambda i: (0, i)),
        ],
        out_specs=pl.BlockSpec((1, 1, _BM), lambda i: (i, 0, 0)),
        out_shape=jax.ShapeDtypeStruct((nb, 1, _BM), jnp.int32),
    )


def _sqrt16(x):
    i = plsc.bitcast(x, jnp.int32)
    i = jnp.int32(0x5F3759DF) - (i >> 1)
    y = plsc.bitcast(i, jnp.float32)
    half_x = 0.5 * x
    for _ in range(3):
        y = y * (1.5 - half_x * y * y)
    return x * y


@functools.cache
def _get_sc_fine(nchunk):
    mesh = plsc.VectorSubcoreMesh(core_axis_name="c", subcore_axis_name="s",
                                  num_cores=_NC, num_subcores=_NS)
    nslot = min(3, nchunk)
    bw = nchunk * _CHUNK
    body = functools.partial(_sc_fine_body, nchunk)
    return pl.kernel(
        body,
        out_type=jax.ShapeDtypeStruct((_NW, 16), jnp.float32),
        mesh=mesh,
        scratch_types=[
            pltpu.VMEM((nchunk, _CHUNK), jnp.int32),
            pltpu.VMEM((_D, bw), jnp.float32),
            pltpu.VMEM((nslot, _CHUNK, _FDP), jnp.float32),
            pltpu.VMEM((16,), jnp.float32),
            *([pltpu.SemaphoreType.DMA] * nslot),
        ],
        compiler_params=pltpu.CompilerParams(needs_layout_passes=False),
    )


def _sc_fine_body(nchunk, feat_hbm, fine_hbm, idx_hbm, out_hbm,
                  idx_v, x_v, rows_v, acc_v, *sems):
    _NSLOT = len(sems)
    wid = lax.axis_index("s") * _NC + lax.axis_index("c")
    bw = nchunk * _CHUNK
    base = wid * bw

    pltpu.sync_copy(idx_hbm.at[wid], idx_v)
    pltpu.sync_copy(feat_hbm.at[:, pl.ds(base, bw)], x_v)

    acc_v[...] = jnp.zeros((16,), jnp.float32)
    lanes = lax.iota(jnp.int32, 16)

    def gather_chunk(j):
        return pltpu.async_copy(fine_hbm.at[idx_v.at[j]], rows_v.at[j % _NSLOT],
                                sems[j % _NSLOT])

    def compute_chunk(j, cp):
        cp.wait()
        slot = j % _NSLOT

        def body(g, carry):
            r0 = g * 16
            row_idx = lanes + r0
            gbase = r0 + (j * _CHUNK)
            slot_idx = jnp.full((16,), slot, jnp.int32)
            zero = jnp.zeros((16,), jnp.float32)

            cmod = idx_v[j, pl.ds(r0, 16)] & (_D - 1)

            def dbody(dh, accs):
                out = list(accs)
                for du in range(2):
                    d = dh * 2 + du
                    col_d = jnp.broadcast_to(d, (16,))
                    xv = x_v[d, pl.ds(gbase, 16)]
                    rot_d = (col_d + cmod) & (_D - 1)
                    for f in range(_F):
                        col = rot_d + (f * _D)
                        fv = plsc.load_gather(rows_v, [slot_idx, row_idx, col])
                        dd = xv - fv
                        out[f] = out[f] + dd * dd
                return tuple(out)

            acc_f = lax.fori_loop(0, _D // 2, dbody, (zero,) * _F)

            m = acc_f[0]
            for f in range(1, _F):
                m = jnp.minimum(m, acc_f[f])
            acc_v[...] = acc_v[...] + _sqrt16(m)
            return carry

        lax.fori_loop(0, _CHUNK // 16, body, 0)

    cps = [gather_chunk(j) for j in range(min(_NSLOT, nchunk))]
    for j in range(nchunk):
        compute_chunk(j, cps[j])
        if j + _NSLOT < nchunk:
            cps.append(gather_chunk(j + _NSLOT))

    pltpu.sync_copy(acc_v, out_hbm.at[wid])


def kernel(features, coarse_prototypes, fine_prototypes):
    cp_pad = jnp.zeros((_CPAD, _D), jnp.float32)
    cp_pad = cp_pad.at[:_C].set(coarse_prototypes)
    feat_t = features.T
    fine_r = fine_prototypes.reshape(_C, _F, _D)
    c_ids = jnp.arange(_C, dtype=jnp.int32)[:, None, None]
    e_ids = jnp.arange(_D, dtype=jnp.int32)[None, None, :]
    src_d = jnp.broadcast_to((e_ids - c_ids) % _D, (_C, _F, _D))
    fine_flat = jnp.take_along_axis(fine_r, src_d, axis=2).reshape(_C, _FD)

    half = _B // 2
    nb = half // _BM
    nchunk_h = half // _NW // _CHUNK
    coarse = _get_coarse_call(nb)
    sc = _get_sc_fine(nchunk_h)
    fts = (feat_t[:, :half], feat_t[:, half:])
    idxs = [coarse(cp_pad, ft).reshape(_NW, nchunk_h, _CHUNK) for ft in fts]
    parts = [sc(ft, fine_flat, ix) for ft, ix in zip(fts, idxs)]
    return (jnp.sum(parts[0]) + jnp.sum(parts[1])) / jnp.float32(_B)


# speedup vs baseline: 1.0151x; 1.0151x over previous
import functools

import jax
import jax.numpy as jnp
from jax import lax
from jax.experimental import pallas as pl
from jax.experimental.pallas import tpu as pltpu
from jax.experimental.pallas import tpu_sc as plsc

_B = 16384
_D = 16
_C = 1000
_CPAD = 1024
_F = 8
_FD = _F * _D
_FDP = 128

_BM = 2048
_NC = 2
_NS = 16
_NW = _NC * _NS
_BW = _B // _NW
_CHUNK = 128
_NCHUNK = _BW // _CHUNK


def _coarse_body(cp_ref, xt_ref, out_ref):
    cp = cp_ref[...]
    xt = xt_ref[...]
    cp2 = -2.0 * cp
    dots2 = lax.dot_general(cp2, xt, (((1,), (0,)), ((), ())),
                            preferred_element_type=jnp.float32)
    row = lax.broadcasted_iota(jnp.int32, (_CPAD, 1), 0)
    b2p = jnp.sum(cp * cp, axis=1, keepdims=True) + jnp.where(
        row >= _C, jnp.float32(1e30), jnp.float32(0.0))
    a2 = jnp.sum(xt * xt, axis=0, keepdims=True)
    d2 = jnp.maximum((dots2 + a2) + b2p, 0.0)
    keys = (lax.bitcast_convert_type(d2, jnp.int32) & jnp.int32(~1023)) | row
    fkeys = lax.bitcast_convert_type(keys, jnp.float32)
    best = lax.bitcast_convert_type(jnp.min(fkeys, axis=0), jnp.int32)
    out_ref[0] = (best & jnp.int32(1023)).reshape(1, _BM)


@functools.cache
def _get_coarse_call(nb, off):
    return pl.pallas_call(
        _coarse_body,
        grid=(nb,),
        in_specs=[
            pl.BlockSpec((_CPAD, _D), lambda i: (0, 0)),
            pl.BlockSpec((_D, _BM), lambda i: (0, i + off)),
        ],
        out_specs=pl.BlockSpec((1, 1, _BM), lambda i: (i, 0, 0)),
        out_shape=jax.ShapeDtypeStruct((nb, 1, _BM), jnp.int32),
    )


def _sqrt16(x):
    i = plsc.bitcast(x, jnp.int32)
    i = jnp.int32(0x5F3759DF) - (i >> 1)
    y = plsc.bitcast(i, jnp.float32)
    half_x = 0.5 * x
    for _ in range(3):
        y = y * (1.5 - half_x * y * y)
    return x * y


@functools.cache
def _get_sc_fine(nchunk, off):
    mesh = plsc.VectorSubcoreMesh(core_axis_name="c", subcore_axis_name="s",
                                  num_cores=_NC, num_subcores=_NS)
    nslot = min(3, nchunk)
    bw = nchunk * _CHUNK
    body = functools.partial(_sc_fine_body, nchunk, off)
    return pl.kernel(
        body,
        out_type=jax.ShapeDtypeStruct((_NW, 16), jnp.float32),
        mesh=mesh,
        scratch_types=[
            pltpu.VMEM((nchunk, _CHUNK), jnp.int32),
            pltpu.VMEM((_D, bw), jnp.float32),
            pltpu.VMEM((nslot, _CHUNK, _FDP), jnp.float32),
            pltpu.VMEM((16,), jnp.float32),
            *([pltpu.SemaphoreType.DMA] * nslot),
        ],
        compiler_params=pltpu.CompilerParams(needs_layout_passes=False),
    )


def _sc_fine_body(nchunk, off, feat_hbm, fine_hbm, idx_hbm, out_hbm,
                  idx_v, x_v, rows_v, acc_v, *sems):
    _NSLOT = len(sems)
    wid = lax.axis_index("s") * _NC + lax.axis_index("c")
    bw = nchunk * _CHUNK
    base = off + wid * bw

    pltpu.sync_copy(idx_hbm.at[wid], idx_v)
    pltpu.sync_copy(feat_hbm.at[:, pl.ds(base, bw)], x_v)

    acc_v[...] = jnp.zeros((16,), jnp.float32)
    lanes = lax.iota(jnp.int32, 16)

    def gather_chunk(j):
        return pltpu.async_copy(fine_hbm.at[idx_v.at[j]], rows_v.at[j % _NSLOT],
                                sems[j % _NSLOT])

    def compute_chunk(j, cp):
        cp.wait()
        slot = j % _NSLOT

        def body(g, carry):
            r0 = g * 16
            row_idx = lanes + r0
            gbase = r0 + (j * _CHUNK)
            slot_idx = jnp.full((16,), slot, jnp.int32)
            zero = jnp.zeros((16,), jnp.float32)

            cmod = idx_v[j, pl.ds(r0, 16)] & (_D - 1)

            def dbody(dh, accs):
                out = list(accs)
                for du in range(2):
                    d = dh * 2 + du
                    col_d = jnp.broadcast_to(d, (16,))
                    xv = x_v[d, pl.ds(gbase, 16)]
                    rot_d = (col_d + cmod) & (_D - 1)
                    for f in range(_F):
                        col = rot_d + (f * _D)
                        fv = plsc.load_gather(rows_v, [slot_idx, row_idx, col])
                        dd = xv - fv
                        out[f] = out[f] + dd * dd
                return tuple(out)

            acc_f = lax.fori_loop(0, _D // 2, dbody, (zero,) * _F)

            m = acc_f[0]
            for f in range(1, _F):
                m = jnp.minimum(m, acc_f[f])
            acc_v[...] = acc_v[...] + _sqrt16(m)
            return carry

        lax.fori_loop(0, _CHUNK // 16, body, 0)

    cps = [gather_chunk(j) for j in range(min(_NSLOT, nchunk))]
    for j in range(nchunk):
        compute_chunk(j, cps[j])
        if j + _NSLOT < nchunk:
            cps.append(gather_chunk(j + _NSLOT))

    pltpu.sync_copy(acc_v, out_hbm.at[wid])


def kernel(features, coarse_prototypes, fine_prototypes):
    cp_pad = jnp.zeros((_CPAD, _D), jnp.float32)
    cp_pad = cp_pad.at[:_C].set(coarse_prototypes)
    feat_t = features.T
    fine_r = fine_prototypes.reshape(_C, _F, _D)
    c_ids = jnp.arange(_C, dtype=jnp.int32)[:, None, None]
    e_ids = jnp.arange(_D, dtype=jnp.int32)[None, None, :]
    src_d = jnp.broadcast_to((e_ids - c_ids) % _D, (_C, _F, _D))
    fine_flat = jnp.take_along_axis(fine_r, src_d, axis=2).reshape(_C, _FD)

    half = _B // 2
    nb = half // _BM
    nchunk_h = half // _NW // _CHUNK
    idxs = [
        _get_coarse_call(nb, h * nb)(cp_pad, feat_t)
        .reshape(_NW, nchunk_h, _CHUNK)
        for h in range(2)
    ]
    parts = [
        _get_sc_fine(nchunk_h, h * half)(feat_t, fine_flat, idxs[h])
        for h in range(2)
    ]
    return (jnp.sum(parts[0]) + jnp.sum(parts[1])) / jnp.float32(_B)


# speedup vs baseline: 1.1635x; 1.1461x over previous
import functools

import jax
import jax.numpy as jnp
from jax import lax
from jax.experimental import pallas as pl
from jax.experimental.pallas import tpu as pltpu
from jax.experimental.pallas import tpu_sc as plsc

_B = 16384
_D = 16
_C = 1000
_CPAD = 1024
_F = 8
_FD = _F * _D
_FDP = 128

_BM = 2048
_NC = 2
_NS = 16
_NW = _NC * _NS
_BW = _B // _NW
_CHUNK = 128
_NCHUNK = _BW // _CHUNK


def _coarse_body(cp_ref, x_ref, out_ref):
    cp = cp_ref[...]
    x = x_ref[...]
    cp2 = -2.0 * cp
    dots2 = lax.dot_general(cp2, x, (((1,), (1,)), ((), ())),
                            preferred_element_type=jnp.float32)
    row = lax.broadcasted_iota(jnp.int32, (_CPAD, 1), 0)
    b2p = jnp.sum(cp * cp, axis=1, keepdims=True) + jnp.where(
        row >= _C, jnp.float32(1e30), jnp.float32(0.0))
    a2 = jnp.sum(x * x, axis=1)[None, :]
    d2 = jnp.maximum((dots2 + a2) + b2p, 0.0)
    keys = (lax.bitcast_convert_type(d2, jnp.int32) & jnp.int32(~1023)) | row
    fkeys = lax.bitcast_convert_type(keys, jnp.float32)
    best = lax.bitcast_convert_type(jnp.min(fkeys, axis=0), jnp.int32)
    out_ref[0] = (best & jnp.int32(1023)).reshape(1, _BM)


@functools.cache
def _get_coarse_call(nb, off):
    return pl.pallas_call(
        _coarse_body,
        grid=(nb,),
        in_specs=[
            pl.BlockSpec((_CPAD, _D), lambda i: (0, 0)),
            pl.BlockSpec((_BM, _D), lambda i: (i + off, 0)),
        ],
        out_specs=pl.BlockSpec((1, 1, _BM), lambda i: (i, 0, 0)),
        out_shape=jax.ShapeDtypeStruct((nb, 1, _BM), jnp.int32),
    )


def _sqrt16(x):
    i = plsc.bitcast(x, jnp.int32)
    i = jnp.int32(0x5F3759DF) - (i >> 1)
    y = plsc.bitcast(i, jnp.float32)
    half_x = 0.5 * x
    for _ in range(3):
        y = y * (1.5 - half_x * y * y)
    return x * y


@functools.cache
def _get_sc_fine(nchunk, off):
    mesh = plsc.VectorSubcoreMesh(core_axis_name="c", subcore_axis_name="s",
                                  num_cores=_NC, num_subcores=_NS)
    nslot = min(3, nchunk)
    bw = nchunk * _CHUNK
    body = functools.partial(_sc_fine_body, nchunk, off)
    return pl.kernel(
        body,
        out_type=jax.ShapeDtypeStruct((_NW, 16), jnp.float32),
        mesh=mesh,
        scratch_types=[
            pltpu.VMEM((nchunk, _CHUNK), jnp.int32),
            pltpu.VMEM((bw, _D), jnp.float32),
            pltpu.VMEM((nslot, _CHUNK, _FDP), jnp.float32),
            pltpu.VMEM((16,), jnp.float32),
            *([pltpu.SemaphoreType.DMA] * nslot),
        ],
        compiler_params=pltpu.CompilerParams(needs_layout_passes=False),
    )


def _sc_fine_body(nchunk, off, feat_hbm, fine_hbm, idx_hbm, out_hbm,
                  idx_v, x_v, rows_v, acc_v, *sems):
    _NSLOT = len(sems)
    wid = lax.axis_index("s") * _NC + lax.axis_index("c")
    bw = nchunk * _CHUNK
    base = off + wid * bw

    pltpu.sync_copy(idx_hbm.at[wid], idx_v)
    pltpu.sync_copy(feat_hbm.at[pl.ds(base, bw)], x_v)

    acc_v[...] = jnp.zeros((16,), jnp.float32)
    lanes = lax.iota(jnp.int32, 16)

    def gather_chunk(j):
        return pltpu.async_copy(fine_hbm.at[idx_v.at[j]], rows_v.at[j % _NSLOT],
                                sems[j % _NSLOT])

    def compute_chunk(j, cp):
        cp.wait()
        slot = j % _NSLOT

        def body(g, carry):
            r0 = g * 16
            row_idx = lanes + r0
            gbase = r0 + (j * _CHUNK)
            slot_idx = jnp.full((16,), slot, jnp.int32)
            zero = jnp.zeros((16,), jnp.float32)

            grow = lanes + gbase

            def dbody(dh, accs):
                out = list(accs)
                for du in range(2):
                    d = dh * 2 + du
                    e = (jnp.broadcast_to(d, (16,)) + lanes) & (_D - 1)
                    xv = plsc.load_gather(x_v, [grow, e])
                    for f in range(_F):
                        col = e + (f * _D)
                        fv = plsc.load_gather(rows_v, [slot_idx, row_idx, col])
                        dd = xv - fv
                        out[f] = out[f] + dd * dd
                return tuple(out)

            acc_f = lax.fori_loop(0, _D // 2, dbody, (zero,) * _F)

            m = acc_f[0]
            for f in range(1, _F):
                m = jnp.minimum(m, acc_f[f])
            acc_v[...] = acc_v[...] + _sqrt16(m)
            return carry

        lax.fori_loop(0, _CHUNK // 16, body, 0)

    cps = [gather_chunk(j) for j in range(min(_NSLOT, nchunk))]
    for j in range(nchunk):
        compute_chunk(j, cps[j])
        if j + _NSLOT < nchunk:
            cps.append(gather_chunk(j + _NSLOT))

    pltpu.sync_copy(acc_v, out_hbm.at[wid])


def kernel(features, coarse_prototypes, fine_prototypes):
    cp_pad = jnp.zeros((_CPAD, _D), jnp.float32)
    cp_pad = cp_pad.at[:_C].set(coarse_prototypes)
    fine_flat = fine_prototypes.reshape(_C, _FD)

    half = _B // 2
    nb = half // _BM
    nchunk_h = half // _NW // _CHUNK
    idxs = [
        _get_coarse_call(nb, h * nb)(cp_pad, features)
        .reshape(_NW, nchunk_h, _CHUNK)
        for h in range(2)
    ]
    parts = [
        _get_sc_fine(nchunk_h, h * half)(features, fine_flat, idxs[h])
        for h in range(2)
    ]
    return (jnp.sum(parts[0]) + jnp.sum(parts[1])) / jnp.float32(_B)
